# baseline (speedup 1.0000x reference)
import jax
import jax.numpy as jnp
from jax.experimental import pallas as pl
from jax.experimental.pallas import tpu as pltpu

_MIN_DIST = 8.0
_STEP_DIST = 0.1
_NUM_BINS = 140
_B, _H, _W = 8, 256, 256
_BLH = 32
_NCH = _B * (_H // _BLH)
_NB = 4


def _body(x_hbm, o_hbm, vin, buf, sin, sout):
    pltpu.async_copy(x_hbm, vin, sin).wait()
    for step in range(_NCH):
        n = step % _NB
        b, hc = divmod(step, _H // _BLH)
        h0 = hc * _BLH
        if step >= _NB:
            bo, hco = divmod(step - _NB, _H // _BLH)
            pltpu.make_async_copy(
                buf.at[n], o_hbm.at[bo, pl.ds(hco * _BLH, _BLH)],
                sout.at[n]).wait()
        x = vin[b, pl.ds(h0, _BLH)]
        idx = ((x - _MIN_DIST) / _STEP_DIST).astype(jnp.int32)
        idx = jnp.minimum(jnp.maximum(idx, 0), _NUM_BINS - 1)
        k = jax.lax.broadcasted_iota(jnp.int32, (_BLH, _W, _NUM_BINS), 2)
        buf[n] = (idx[..., None] == k).astype(jnp.float32)
        pltpu.async_copy(buf.at[n], o_hbm.at[b, pl.ds(h0, _BLH)], sout.at[n])
    for step in range(_NCH - _NB, _NCH):
        n = step % _NB
        bo, hco = divmod(step, _H // _BLH)
        pltpu.make_async_copy(
            buf.at[n], o_hbm.at[bo, pl.ds(hco * _BLH, _BLH)],
            sout.at[n]).wait()


def kernel(ipt, table):
    del table
    return pl.pallas_call(
        _body,
        in_specs=[pl.BlockSpec(memory_space=pl.ANY)],
        out_specs=pl.BlockSpec(memory_space=pl.ANY),
        out_shape=jax.ShapeDtypeStruct((_B, _H, _W, _NUM_BINS), jnp.float32),
        scratch_shapes=[
            pltpu.VMEM((_B, _H, _W), jnp.float32),
            pltpu.VMEM((_NB, _BLH, _W, _NUM_BINS), jnp.float32),
            pltpu.SemaphoreType.DMA,
            pltpu.SemaphoreType.DMA((_NB,)),
        ],
    )(ipt)
